# baseline (speedup 1.0000x reference)
import functools

import jax
import jax.numpy as jnp
from jax import lax
from jax.experimental import pallas as pl
from jax.experimental.pallas import tpu as pltpu
from jax.experimental.pallas import tpu_sc as plsc

VOCAB = 100000
EMB = 128
BATCH = 1024
CTX = 20

NC, NS = 2, 16
NW = NC * NS
N_IDX = BATCH * CTX
B_PER_W = N_IDX // NW
IDX_CHUNK = 128
CHUNKS = B_PER_W // IDX_CHUNK

VT = 2048


def _sc_gather(table, idx_flat):
    mesh = plsc.VectorSubcoreMesh(core_axis_name="c", subcore_axis_name="s")

    @functools.partial(
        pl.kernel,
        out_type=jax.ShapeDtypeStruct((N_IDX, EMB), jnp.float32),
        mesh=mesh,
        scratch_types=[
            pltpu.VMEM((B_PER_W,), jnp.int32),
            pltpu.VMEM((B_PER_W, EMB), jnp.float32),
            pltpu.SemaphoreType.DMA,
        ],
    )
    def gather_kernel(table_hbm, idx_hbm, out_hbm, idx_v, rows_v, sem):
        wid = lax.axis_index("s") * NC + lax.axis_index("c")
        pltpu.sync_copy(idx_hbm.at[pl.ds(wid * B_PER_W, B_PER_W)], idx_v)
        copies = []
        for j in range(CHUNKS):
            copies.append(
                pltpu.async_copy(
                    table_hbm.at[idx_v.at[pl.ds(j * IDX_CHUNK, IDX_CHUNK)]],
                    rows_v.at[pl.ds(j * IDX_CHUNK, IDX_CHUNK)],
                    sem,
                )
            )
        for c in copies:
            c.wait()
        pltpu.sync_copy(rows_v, out_hbm.at[pl.ds(wid * B_PER_W, B_PER_W)])

    return gather_kernel(table, idx_flat)


def _tc_project(gathered3, w, b2d):
    grid = (pl.cdiv(VOCAB, VT),)

    def body(g_ref, w_ref, b_ref, o_ref, emb_ref):
        @pl.when(pl.program_id(0) == 0)
        def _():
            emb_ref[...] = jnp.sum(g_ref[...], axis=1).astype(jnp.bfloat16)

        wt = w_ref[...].astype(jnp.bfloat16)
        acc = lax.dot_general(
            emb_ref[...],
            wt,
            dimension_numbers=(((1,), (1,)), ((), ())),
            preferred_element_type=jnp.float32,
        )
        o_ref[...] = acc + b_ref[...]

    return pl.pallas_call(
        body,
        grid=grid,
        in_specs=[
            pl.BlockSpec((BATCH, CTX, EMB), lambda i: (0, 0, 0)),
            pl.BlockSpec((VT, EMB), lambda i: (i, 0)),
            pl.BlockSpec((1, VT), lambda i: (0, i)),
        ],
        out_specs=pl.BlockSpec((BATCH, VT), lambda i: (0, i)),
        out_shape=jax.ShapeDtypeStruct((BATCH, VOCAB), jnp.float32),
        scratch_shapes=[pltpu.VMEM((BATCH, EMB), jnp.bfloat16)],
    )(gathered3, w, b2d)


def kernel(X, W_emb, W1_w, W1_b):
    gathered = _sc_gather(W_emb, X.reshape(N_IDX))
    gathered3 = gathered.reshape(BATCH, CTX, EMB)
    return _tc_project(gathered3, W1_w, W1_b.reshape(1, VOCAB))


# speedup vs baseline: 1.0281x; 1.0281x over previous
import functools

import jax
import jax.numpy as jnp
import numpy as np
from jax import lax
from jax.experimental import pallas as pl
from jax.experimental.pallas import tpu as pltpu
from jax.experimental.pallas import tpu_sc as plsc

VOCAB = 100000
EMB = 128
BATCH = 1024
CTX = 20

NC, NS = 2, 16
NW = NC * NS
N_IDX = BATCH * CTX
B_PER_W = N_IDX // NW
ROWS_PER_W = BATCH // NW
ROWS_PER_C = BATCH // NC
IDX_CHUNK = 128
CHUNKS = B_PER_W // IDX_CHUNK
SEG_ROWS = 8

VT = 2048

_SEG_NP = np.zeros((NW, SEG_ROWS, IDX_CHUNK), np.int32)
_seg_flat = ((np.arange(N_IDX) // CTX) % ROWS_PER_C).astype(np.int32)
_SEG_NP[:, :CHUNKS, :] = _seg_flat.reshape(NW, CHUNKS, IDX_CHUNK)


def _sc_embed(table, idx_flat, seg_map, zeros_blk):
    mesh = plsc.VectorSubcoreMesh(core_axis_name="c", subcore_axis_name="s")

    @functools.partial(
        pl.kernel,
        out_type=jax.ShapeDtypeStruct((BATCH, EMB), jnp.float32),
        mesh=mesh,
        scratch_types=[
            pltpu.VMEM((B_PER_W,), jnp.int32),
            pltpu.VMEM((SEG_ROWS, IDX_CHUNK), jnp.int32),
            pltpu.VMEM((B_PER_W, EMB), jnp.float32),
            pltpu.VMEM_SHARED((ROWS_PER_C, EMB), jnp.float32),
            pltpu.SemaphoreType.DMA,
        ],
    )
    def embed_kernel(table_hbm, idx_hbm, seg_hbm, zeros_hbm, out_hbm,
                     idx_v, seg_v, rows_v, emb_sh, sem):
        cid = lax.axis_index("c")
        sid = lax.axis_index("s")
        wid = cid * NS + sid
        pltpu.sync_copy(idx_hbm.at[pl.ds(wid * B_PER_W, B_PER_W)], idx_v)
        pltpu.sync_copy(seg_hbm.at[wid], seg_v)

        @pl.when(sid == 0)
        def _():
            pltpu.sync_copy(zeros_hbm, emb_sh)

        copies = []
        for j in range(CHUNKS):
            copies.append(
                pltpu.async_copy(
                    table_hbm.at[idx_v.at[pl.ds(j * IDX_CHUNK, IDX_CHUNK)]],
                    rows_v.at[pl.ds(j * IDX_CHUNK, IDX_CHUNK)],
                    sem,
                )
            )
        plsc.subcore_barrier()
        for j, c in enumerate(copies):
            c.wait()
            pltpu.sync_copy(
                rows_v.at[pl.ds(j * IDX_CHUNK, IDX_CHUNK)],
                emb_sh.at[seg_v.at[j]],
                add=True,
            )
        plsc.subcore_barrier()
        pltpu.sync_copy(
            emb_sh.at[pl.ds(sid * ROWS_PER_W, ROWS_PER_W)],
            out_hbm.at[pl.ds(wid * ROWS_PER_W, ROWS_PER_W)],
        )

    return embed_kernel(table, idx_flat, seg_map, zeros_blk)


def _tc_project(emb, w, b2d):
    grid = (pl.cdiv(VOCAB, VT),)

    def body(emb_ref, w_ref, b_ref, o_ref, ebf_ref):
        @pl.when(pl.program_id(0) == 0)
        def _():
            ebf_ref[...] = emb_ref[...].astype(jnp.bfloat16)

        wt = w_ref[...].astype(jnp.bfloat16)
        acc = lax.dot_general(
            ebf_ref[...],
            wt,
            dimension_numbers=(((1,), (1,)), ((), ())),
            preferred_element_type=jnp.float32,
        )
        o_ref[...] = acc + b_ref[...]

    return pl.pallas_call(
        body,
        grid=grid,
        in_specs=[
            pl.BlockSpec((BATCH, EMB), lambda i: (0, 0)),
            pl.BlockSpec((VT, EMB), lambda i: (i, 0)),
            pl.BlockSpec((1, VT), lambda i: (0, i)),
        ],
        out_specs=pl.BlockSpec((BATCH, VT), lambda i: (0, i)),
        out_shape=jax.ShapeDtypeStruct((BATCH, VOCAB), jnp.float32),
        scratch_shapes=[pltpu.VMEM((BATCH, EMB), jnp.bfloat16)],
        compiler_params=pltpu.CompilerParams(
            dimension_semantics=("arbitrary",),
        ),
    )(emb, w, b2d)


def kernel(X, W_emb, W1_w, W1_b):
    seg_map = jnp.asarray(_SEG_NP)
    zeros_blk = jnp.zeros((ROWS_PER_C, EMB), jnp.float32)
    emb = _sc_embed(W_emb, X.reshape(N_IDX), seg_map, zeros_blk)
    return _tc_project(emb, W1_w, W1_b.reshape(1, VOCAB))


# speedup vs baseline: 2.4690x; 2.4014x over previous
import functools

import jax
import jax.numpy as jnp
import numpy as np
from jax import lax
from jax.experimental import pallas as pl
from jax.experimental.pallas import tpu as pltpu
from jax.experimental.pallas import tpu_sc as plsc

VOCAB = 100000
EMB = 128
BATCH = 1024
CTX = 20

NC, NS = 2, 16
NW = NC * NS
N_IDX = BATCH * CTX
B_PER_W = N_IDX // NW
ROWS_PER_W = BATCH // NW
ROWS_PER_C = BATCH // NC
IDX_CHUNK = 128
CHUNKS = B_PER_W // IDX_CHUNK
SEG_ROWS = 8

VT = 4096

_SEG_NP = np.zeros((NW, SEG_ROWS, IDX_CHUNK), np.int32)
_seg_flat = ((np.arange(N_IDX) // CTX) % ROWS_PER_C).astype(np.int32)
_SEG_NP[:, :CHUNKS, :] = _seg_flat.reshape(NW, CHUNKS, IDX_CHUNK)


def _sc_embed(table, idx_flat, seg_map, zeros_blk):
    mesh = plsc.VectorSubcoreMesh(core_axis_name="c", subcore_axis_name="s")

    @functools.partial(
        pl.kernel,
        out_type=jax.ShapeDtypeStruct((BATCH, EMB), jnp.float32),
        mesh=mesh,
        scratch_types=[
            pltpu.VMEM((B_PER_W,), jnp.int32),
            pltpu.VMEM((SEG_ROWS, IDX_CHUNK), jnp.int32),
            pltpu.VMEM((B_PER_W, EMB), jnp.float32),
            pltpu.VMEM_SHARED((ROWS_PER_C, EMB), jnp.float32),
            pltpu.SemaphoreType.DMA,
        ],
    )
    def embed_kernel(table_hbm, idx_hbm, seg_hbm, zeros_hbm, out_hbm,
                     idx_v, seg_v, rows_v, emb_sh, sem):
        cid = lax.axis_index("c")
        sid = lax.axis_index("s")
        wid = cid * NS + sid
        pltpu.sync_copy(idx_hbm.at[pl.ds(wid * B_PER_W, B_PER_W)], idx_v)
        pltpu.sync_copy(seg_hbm.at[wid], seg_v)

        @pl.when(sid == 0)
        def _():
            pltpu.sync_copy(zeros_hbm, emb_sh)

        copies = []
        for j in range(CHUNKS):
            copies.append(
                pltpu.async_copy(
                    table_hbm.at[idx_v.at[pl.ds(j * IDX_CHUNK, IDX_CHUNK)]],
                    rows_v.at[pl.ds(j * IDX_CHUNK, IDX_CHUNK)],
                    sem,
                )
            )
        plsc.subcore_barrier()
        for j, c in enumerate(copies):
            c.wait()
            pltpu.sync_copy(
                rows_v.at[pl.ds(j * IDX_CHUNK, IDX_CHUNK)],
                emb_sh.at[seg_v.at[j]],
                add=True,
            )
        plsc.subcore_barrier()
        pltpu.sync_copy(
            emb_sh.at[pl.ds(sid * ROWS_PER_W, ROWS_PER_W)],
            out_hbm.at[pl.ds(wid * ROWS_PER_W, ROWS_PER_W)],
        )

    return embed_kernel(table, idx_flat, seg_map, zeros_blk)


def _tc_project_t(emb, w, bcol):
    grid = (pl.cdiv(VOCAB, VT),)

    def body(emb_ref, w_ref, b_ref, o_ref, ebf_ref):
        @pl.when(pl.program_id(0) == 0)
        def _():
            ebf_ref[...] = emb_ref[...].astype(jnp.bfloat16)

        wt = w_ref[...].astype(jnp.bfloat16)
        acc = lax.dot_general(
            wt,
            ebf_ref[...],
            dimension_numbers=(((1,), (1,)), ((), ())),
            preferred_element_type=jnp.float32,
        )
        o_ref[...] = acc + b_ref[...]

    return pl.pallas_call(
        body,
        grid=grid,
        in_specs=[
            pl.BlockSpec((BATCH, EMB), lambda i: (0, 0)),
            pl.BlockSpec((VT, EMB), lambda i: (i, 0)),
            pl.BlockSpec((VT, 1), lambda i: (i, 0)),
        ],
        out_specs=pl.BlockSpec((VT, BATCH), lambda i: (i, 0)),
        out_shape=jax.ShapeDtypeStruct((VOCAB, BATCH), jnp.float32),
        scratch_shapes=[pltpu.VMEM((BATCH, EMB), jnp.bfloat16)],
        compiler_params=pltpu.CompilerParams(
            dimension_semantics=("arbitrary",),
        ),
    )(emb, w, bcol)


def kernel(X, W_emb, W1_w, W1_b):
    seg_map = jnp.asarray(_SEG_NP)
    zeros_blk = jnp.zeros((ROWS_PER_C, EMB), jnp.float32)
    emb = _sc_embed(W_emb, X.reshape(N_IDX), seg_map, zeros_blk)
    out_t = _tc_project_t(emb, W1_w, W1_b.reshape(VOCAB, 1))
    return out_t.T
